# baseline (speedup 1.0000x reference)
import functools

import jax
import jax.numpy as jnp
from jax import lax
from jax.experimental import pallas as pl
from jax.experimental.pallas import tpu as pltpu
from jax.experimental.pallas import tpu_sc as plsc


def _broadcast_rows(B, S, D, dtype):
    info = plsc.get_sparse_core_info()
    NC, NS = info.num_cores, info.num_subcores
    NW = NC * NS
    rows_per_w = S // NW
    CH = min(32, rows_per_w)
    n_ch = rows_per_w // CH

    mesh = plsc.VectorSubcoreMesh(core_axis_name="c", subcore_axis_name="s")

    @functools.partial(
        pl.kernel,
        mesh=mesh,
        out_type=jax.ShapeDtypeStruct((B, S, D), dtype),
        scratch_types=[
            pltpu.VMEM((CH, D), dtype),
            pltpu.VMEM((CH, D), dtype),
            pltpu.SemaphoreType.DMA,
            pltpu.SemaphoreType.DMA,
        ],
    )
    def k(table_hbm, out_hbm, buf0, buf1, wsem0, wsem1):
        wid = lax.axis_index("s") * NC + lax.axis_index("c")
        base = wid * rows_per_w
        bufs = (buf0, buf1)
        wsems = (wsem0, wsem1)
        pending = [[], []]
        for c in range(n_ch):
            cur = c % 2
            for h in pending[cur]:
                h.wait()
            pending[cur] = []
            r0 = base + c * CH
            pltpu.sync_copy(table_hbm.at[pl.ds(r0, CH)], bufs[cur])
            for b in range(B):
                pending[cur].append(
                    pltpu.async_copy(bufs[cur], out_hbm.at[b, pl.ds(r0, CH)],
                                     wsems[cur]))
        for lst in pending:
            for h in lst:
                h.wait()

    return k


def _broadcast_rows_tc(B, S, D, dtype, s0, s_len):
    S_BLK = 512
    n_s = s_len // S_BLK

    def body(emb_ref, out_ref):
        blk = emb_ref[...][None]
        for b in range(B):
            out_ref[pl.ds(b, 1)] = blk

    return pl.pallas_call(
        body,
        grid=(n_s,),
        in_specs=[pl.BlockSpec((S_BLK, D), lambda i: (s0 // S_BLK + i, 0))],
        out_specs=pl.BlockSpec((B, S_BLK, D), lambda i: (0, i, 0)),
        out_shape=jax.ShapeDtypeStruct((B, s_len, D), dtype),
    )


def _broadcast_rows_sc(B, s_len, D, dtype, s0, S_total):
    info = plsc.get_sparse_core_info()
    NC, NS = info.num_cores, info.num_subcores
    NW = NC * NS
    rows_per_w = s_len // NW
    CH = min(32, rows_per_w)
    n_ch = rows_per_w // CH

    mesh = plsc.VectorSubcoreMesh(core_axis_name="c", subcore_axis_name="s")

    @functools.partial(
        pl.kernel,
        mesh=mesh,
        out_type=jax.ShapeDtypeStruct((B, s_len, D), dtype),
        scratch_types=[
            pltpu.VMEM((CH, D), dtype),
            pltpu.VMEM((CH, D), dtype),
            pltpu.SemaphoreType.DMA,
            pltpu.SemaphoreType.DMA,
        ],
    )
    def k(table_hbm, out_hbm, buf0, buf1, wsem0, wsem1):
        wid = lax.axis_index("s") * NC + lax.axis_index("c")
        base = wid * rows_per_w
        bufs = (buf0, buf1)
        wsems = (wsem0, wsem1)
        pending = [[], []]
        for c in range(n_ch):
            cur = c % 2
            for h in pending[cur]:
                h.wait()
            pending[cur] = []
            r0 = base + c * CH
            pltpu.sync_copy(table_hbm.at[pl.ds(s0 + r0, CH)], bufs[cur])
            for b in range(B):
                pending[cur].append(
                    pltpu.async_copy(bufs[cur], out_hbm.at[b, pl.ds(r0, CH)],
                                     wsems[cur]))
        for lst in pending:
            for h in lst:
                h.wait()

    return k


def kernel(input_ids, embeddings):
    B, S = input_ids.shape
    M, D = embeddings.shape
    B_SC = 1
    sc_fn = _broadcast_rows_sc(B_SC, S, D, embeddings.dtype, 0, S)
    tc_fn = _broadcast_rows_tc(B - B_SC, S, D, embeddings.dtype, 0, S)
    out_sc = sc_fn(embeddings)
    out_tc = tc_fn(embeddings)
    return jnp.concatenate([out_sc, out_tc], axis=0)


# speedup vs baseline: 1.9856x; 1.9856x over previous
import functools

import jax
import jax.numpy as jnp
from jax import lax
from jax.experimental import pallas as pl
from jax.experimental.pallas import tpu as pltpu
from jax.experimental.pallas import tpu_sc as plsc


def _broadcast_rows(B, S, D, dtype):
    info = plsc.get_sparse_core_info()
    NC, NS = info.num_cores, info.num_subcores
    NW = NC * NS
    rows_per_w = S // NW
    CH = min(32, rows_per_w)
    n_ch = rows_per_w // CH

    mesh = plsc.VectorSubcoreMesh(core_axis_name="c", subcore_axis_name="s")

    @functools.partial(
        pl.kernel,
        mesh=mesh,
        out_type=jax.ShapeDtypeStruct((B, S, D), dtype),
        scratch_types=[
            pltpu.VMEM((CH, D), dtype),
            pltpu.VMEM((CH, D), dtype),
            pltpu.SemaphoreType.DMA,
            pltpu.SemaphoreType.DMA,
        ],
    )
    def k(table_hbm, out_hbm, buf0, buf1, wsem0, wsem1):
        wid = lax.axis_index("s") * NC + lax.axis_index("c")
        base = wid * rows_per_w
        bufs = (buf0, buf1)
        wsems = (wsem0, wsem1)
        pending = [[], []]
        for c in range(n_ch):
            cur = c % 2
            for h in pending[cur]:
                h.wait()
            pending[cur] = []
            r0 = base + c * CH
            pltpu.sync_copy(table_hbm.at[pl.ds(r0, CH)], bufs[cur])
            for b in range(B):
                pending[cur].append(
                    pltpu.async_copy(bufs[cur], out_hbm.at[b, pl.ds(r0, CH)],
                                     wsems[cur]))
        for lst in pending:
            for h in lst:
                h.wait()

    return k


def _broadcast_rows_tc(B, S, D, dtype, s0, s_len):
    S_BLK = 512
    n_s = s_len // S_BLK

    def body(emb_ref, out_ref):
        blk = emb_ref[...][None]
        for b in range(B):
            out_ref[pl.ds(b, 1)] = blk

    return pl.pallas_call(
        body,
        grid=(n_s,),
        in_specs=[pl.BlockSpec((S_BLK, D), lambda i: (s0 // S_BLK + i, 0))],
        out_specs=pl.BlockSpec((B, S_BLK, D), lambda i: (0, i, 0)),
        out_shape=jax.ShapeDtypeStruct((B, s_len, D), dtype),
    )


def _broadcast_rows_sc(B, s_len, D, dtype, s0, S_total):
    info = plsc.get_sparse_core_info()
    NC, NS = info.num_cores, info.num_subcores
    NW = NC * NS
    rows_per_w = s_len // NW
    CH = min(32, rows_per_w)
    n_ch = rows_per_w // CH

    mesh = plsc.VectorSubcoreMesh(core_axis_name="c", subcore_axis_name="s")

    @functools.partial(
        pl.kernel,
        mesh=mesh,
        out_type=jax.ShapeDtypeStruct((B, s_len, D), dtype),
        scratch_types=[
            pltpu.VMEM((CH, D), dtype),
            pltpu.VMEM((CH, D), dtype),
            pltpu.SemaphoreType.DMA,
            pltpu.SemaphoreType.DMA,
        ],
    )
    def k(table_hbm, out_hbm, buf0, buf1, wsem0, wsem1):
        wid = lax.axis_index("s") * NC + lax.axis_index("c")
        base = wid * rows_per_w
        bufs = (buf0, buf1)
        wsems = (wsem0, wsem1)
        pending = [[], []]
        for c in range(n_ch):
            cur = c % 2
            for h in pending[cur]:
                h.wait()
            pending[cur] = []
            r0 = base + c * CH
            pltpu.sync_copy(table_hbm.at[pl.ds(s0 + r0, CH)], bufs[cur])
            for b in range(B):
                pending[cur].append(
                    pltpu.async_copy(bufs[cur], out_hbm.at[b, pl.ds(r0, CH)],
                                     wsems[cur]))
        for lst in pending:
            for h in lst:
                h.wait()

    return k


def _broadcast_rows_sc3(B, S, D, dtype):
    info = plsc.get_sparse_core_info()
    NC, NS = info.num_cores, info.num_subcores
    NW = NC * NS
    rows_per_w = S // NW
    CH = min(32, rows_per_w)
    n_ch = rows_per_w // CH
    NB = min(3, n_ch)

    mesh = plsc.VectorSubcoreMesh(core_axis_name="c", subcore_axis_name="s")

    @functools.partial(
        pl.kernel,
        mesh=mesh,
        out_type=jax.ShapeDtypeStruct((B, S, D), dtype),
        scratch_types=(
            [pltpu.VMEM((CH, D), dtype)] * NB
            + [pltpu.SemaphoreType.DMA] * (2 * NB)
        ),
    )
    def k(table_hbm, out_hbm, *scratch):
        bufs = scratch[:NB]
        rsems = scratch[NB:2 * NB]
        wsems = scratch[2 * NB:]
        wid = lax.axis_index("s") * NC + lax.axis_index("c")
        base = wid * rows_per_w

        def read(c):
            r0 = base + c * CH
            return pltpu.async_copy(table_hbm.at[pl.ds(r0, CH)], bufs[c % NB],
                                    rsems[c % NB])

        rh = {}
        wh = {}
        for c in range(min(NB - 1, n_ch)):
            rh[c] = read(c)
        for c in range(n_ch):
            cur = c % NB
            if c not in rh:
                rh[c] = read(c)
            rh[c].wait()
            r0 = base + c * CH
            wh[cur] = [
                pltpu.async_copy(bufs[cur], out_hbm.at[b, pl.ds(r0, CH)],
                                 wsems[cur])
                for b in range(B)
            ]
            nb = c + NB - 1
            if nb < n_ch:
                for h in wh.get(nb % NB, []):
                    h.wait()
                rh[nb] = read(nb)
        for lst in wh.values():
            for h in lst:
                h.wait()

    return k


def kernel(input_ids, embeddings):
    B, S = input_ids.shape
    M, D = embeddings.shape
    fn = _broadcast_rows_sc3(B, S, D, embeddings.dtype)
    return fn(embeddings)
